# speedup vs baseline: 1.6230x; 1.6230x over previous
import functools

import jax
import jax.numpy as jnp
from jax import lax
from jax.experimental import pallas as pl
from jax.experimental.pallas import tpu as pltpu
from jax.experimental.pallas import tpu_sc as plsc

_INFO = plsc.get_sparse_core_info()
_NC = _INFO.num_cores
_NS = _INFO.num_subcores
_NW = _NC * _NS

_CHUNK = 32


def _make_gather(n_tokens: int, dim: int):
  n_per_w = n_tokens // _NW
  n_chunks = n_per_w // _CHUNK
  mesh = plsc.VectorSubcoreMesh(core_axis_name="c", subcore_axis_name="s")

  @functools.partial(
      pl.kernel,
      out_type=jax.ShapeDtypeStruct((n_tokens, dim), jnp.float32),
      mesh=mesh,
      scratch_types=[
          pltpu.VMEM((n_per_w,), jnp.int32),
          pltpu.VMEM((_CHUNK, dim), jnp.float32),
          pltpu.SemaphoreType.DMA,
      ],
  )
  def gather_kernel(ids_hbm, table_hbm, out_hbm, idx_v, rows_v, sem):
    wid = lax.axis_index("s") * _NC + lax.axis_index("c")
    base = wid * n_per_w
    pltpu.sync_copy(ids_hbm.at[pl.ds(base, n_per_w)], idx_v)

    def chunk_body(g, carry):
      off = g * _CHUNK
      pltpu.async_copy(
          table_hbm.at[idx_v.at[pl.ds(off, _CHUNK)]], rows_v, sem
      ).wait()
      pltpu.sync_copy(rows_v, out_hbm.at[pl.ds(base + off, _CHUNK)])
      return carry

    lax.fori_loop(0, n_chunks, chunk_body, 0)

  return gather_kernel


def kernel(input_ids, input_mask, table):
  del input_mask
  b, s = input_ids.shape
  vocab, dim = table.shape
  ids_flat = input_ids.reshape(-1).astype(jnp.int32)
  out = _make_gather(b * s, dim)(ids_flat, table)
  return out.reshape(b, s, dim)


# speedup vs baseline: 1.7629x; 1.0862x over previous
import functools

import jax
import jax.numpy as jnp
from jax import lax
from jax.experimental import pallas as pl
from jax.experimental.pallas import tpu as pltpu
from jax.experimental.pallas import tpu_sc as plsc

_INFO = plsc.get_sparse_core_info()
_NC = _INFO.num_cores
_NS = _INFO.num_subcores
_NW = _NC * _NS

_CHUNK = 16
_NBUF = 2


def _make_gather(n_tokens: int, dim: int):
  n_per_w = n_tokens // _NW
  n_chunks = n_per_w // _CHUNK
  assert n_chunks % _NBUF == 0 and n_chunks >= 2 * _NBUF
  mesh = plsc.VectorSubcoreMesh(core_axis_name="c", subcore_axis_name="s")

  @functools.partial(
      pl.kernel,
      out_type=jax.ShapeDtypeStruct((n_tokens, dim), jnp.float32),
      mesh=mesh,
      scratch_types=[
          pltpu.VMEM((n_per_w,), jnp.int32),
          [pltpu.VMEM((_CHUNK, dim), jnp.float32) for _ in range(_NBUF)],
          [pltpu.SemaphoreType.DMA for _ in range(_NBUF)],
          [pltpu.SemaphoreType.DMA for _ in range(_NBUF)],
      ],
  )
  def gather_kernel(ids_hbm, table_hbm, out_hbm, idx_v, rows, gsem, ssem):
    wid = lax.axis_index("s") * _NC + lax.axis_index("c")
    base = wid * n_per_w
    pltpu.sync_copy(ids_hbm.at[pl.ds(base, n_per_w)], idx_v)

    def start_gather(chunk, b):
      pltpu.async_copy(
          table_hbm.at[idx_v.at[pl.ds(chunk * _CHUNK, _CHUNK)]], rows[b],
          gsem[b])

    def wait_gather(b):
      pltpu.make_async_copy(
          table_hbm.at[pl.ds(0, _CHUNK)], rows[b], gsem[b]).wait()

    def start_store(chunk, b):
      pltpu.async_copy(
          rows[b], out_hbm.at[pl.ds(base + chunk * _CHUNK, _CHUNK)], ssem[b])

    def wait_store(b):
      pltpu.make_async_copy(
          rows[b], out_hbm.at[pl.ds(0, _CHUNK)], ssem[b]).wait()

    for b in range(_NBUF):
      start_gather(b, b)

    def body(go, carry):
      for b in range(_NBUF):
        g = go * _NBUF + b
        wait_gather(b)
        start_store(g, b)
        @pl.when(go < (n_chunks // _NBUF) - 1)
        def _():
          wait_store(b)
          start_gather(g + _NBUF, b)
      return carry

    lax.fori_loop(0, n_chunks // _NBUF, body, 0)
    for b in range(_NBUF):
      wait_store(b)

  return gather_kernel


def kernel(input_ids, input_mask, table):
  del input_mask
  b, s = input_ids.shape
  vocab, dim = table.shape
  ids_flat = input_ids.reshape(-1).astype(jnp.int32)
  out = _make_gather(b * s, dim)(ids_flat, table)
  return out.reshape(b, s, dim)


# speedup vs baseline: 1.7788x; 1.0090x over previous
import functools

import jax
import jax.numpy as jnp
from jax import lax
from jax.experimental import pallas as pl
from jax.experimental.pallas import tpu as pltpu
from jax.experimental.pallas import tpu_sc as plsc

_INFO = plsc.get_sparse_core_info()
_NC = _INFO.num_cores
_NS = _INFO.num_subcores
_NW = _NC * _NS

_CHUNK = 8
_NBUF = 4


def _make_gather(n_tokens: int, dim: int):
  n_per_w = n_tokens // _NW
  n_chunks = n_per_w // _CHUNK
  assert n_chunks % _NBUF == 0 and n_chunks >= 2 * _NBUF
  mesh = plsc.VectorSubcoreMesh(core_axis_name="c", subcore_axis_name="s")

  @functools.partial(
      pl.kernel,
      out_type=jax.ShapeDtypeStruct((n_tokens, dim), jnp.float32),
      mesh=mesh,
      scratch_types=[
          pltpu.VMEM((n_per_w,), jnp.int32),
          [pltpu.VMEM((_CHUNK, dim), jnp.float32) for _ in range(_NBUF)],
          [pltpu.SemaphoreType.DMA for _ in range(_NBUF)],
          [pltpu.SemaphoreType.DMA for _ in range(_NBUF)],
      ],
  )
  def gather_kernel(ids_hbm, table_hbm, out_hbm, idx_v, rows, gsem, ssem):
    wid = lax.axis_index("s") * _NC + lax.axis_index("c")
    base = wid * n_per_w
    pltpu.sync_copy(ids_hbm.at[pl.ds(base, n_per_w)], idx_v)

    def start_gather(chunk, b):
      pltpu.async_copy(
          table_hbm.at[idx_v.at[pl.ds(chunk * _CHUNK, _CHUNK)]], rows[b],
          gsem[b])

    def wait_gather(b):
      pltpu.make_async_copy(
          table_hbm.at[pl.ds(0, _CHUNK)], rows[b], gsem[b]).wait()

    def start_store(chunk, b):
      pltpu.async_copy(
          rows[b], out_hbm.at[pl.ds(base + chunk * _CHUNK, _CHUNK)], ssem[b])

    def wait_store(b):
      pltpu.make_async_copy(
          rows[b], out_hbm.at[pl.ds(0, _CHUNK)], ssem[b]).wait()

    for b in range(_NBUF):
      start_gather(b, b)

    def body(go, carry):
      for b in range(_NBUF):
        g = go * _NBUF + b
        wait_gather(b)
        start_store(g, b)
        @pl.when(go < (n_chunks // _NBUF) - 1)
        def _():
          wait_store(b)
          start_gather(g + _NBUF, b)
      return carry

    lax.fori_loop(0, n_chunks // _NBUF, body, 0)
    for b in range(_NBUF):
      wait_store(b)

  return gather_kernel


def kernel(input_ids, input_mask, table):
  del input_mask
  b, s = input_ids.shape
  vocab, dim = table.shape
  ids_flat = input_ids.reshape(-1).astype(jnp.int32)
  out = _make_gather(b * s, dim)(ids_flat, table)
  return out.reshape(b, s, dim)


# speedup vs baseline: 1.7972x; 1.0103x over previous
import functools

import jax
import jax.numpy as jnp
from jax import lax
from jax.experimental import pallas as pl
from jax.experimental.pallas import tpu as pltpu
from jax.experimental.pallas import tpu_sc as plsc

_INFO = plsc.get_sparse_core_info()
_NC = _INFO.num_cores
_NS = _INFO.num_subcores
_NW = _NC * _NS

_CHUNK = 8
_NBUF = 4


def _make_gather(n_tokens: int, dim: int):
  n_per_w = n_tokens // _NW
  n_chunks = n_per_w // _CHUNK
  assert n_chunks % _NBUF == 0 and n_chunks >= 2 * _NBUF
  mesh = plsc.VectorSubcoreMesh(core_axis_name="c", subcore_axis_name="s")

  @functools.partial(
      pl.kernel,
      out_type=jax.ShapeDtypeStruct((n_tokens, dim), jnp.float32),
      mesh=mesh,
      scratch_types=[
          pltpu.VMEM((n_per_w,), jnp.int32),
          [pltpu.VMEM((_CHUNK, dim), jnp.float32) for _ in range(_NBUF)],
          pltpu.VMEM_SHARED((_NS, 2, _CHUNK, dim), jnp.float32),
          [pltpu.SemaphoreType.DMA for _ in range(_NBUF)],
          [pltpu.SemaphoreType.DMA for _ in range(_NBUF)],
          [pltpu.SemaphoreType.DMA for _ in range(2)],
      ],
  )
  def gather_kernel(ids_hbm, table_hbm, out_hbm, idx_v, rows, stage,
                    gsem, csem, hsem):
    sid = lax.axis_index("s")
    wid = sid * _NC + lax.axis_index("c")
    base = wid * n_per_w
    pltpu.sync_copy(ids_hbm.at[pl.ds(base, n_per_w)], idx_v)

    def start_gather(chunk, b):
      pltpu.async_copy(
          table_hbm.at[idx_v.at[pl.ds(chunk * _CHUNK, _CHUNK)]], rows[b],
          gsem[b])

    def wait_gather(b):
      pltpu.make_async_copy(
          table_hbm.at[pl.ds(0, _CHUNK)], rows[b], gsem[b]).wait()

    def start_stage(b):
      pltpu.async_copy(rows[b], stage.at[sid, b % 2], csem[b])

    def wait_stage(b):
      pltpu.make_async_copy(rows[b], stage.at[sid, b % 2], csem[b]).wait()

    def start_store(chunk, b):
      pltpu.async_copy(
          stage.at[sid, b % 2], out_hbm.at[pl.ds(base + chunk * _CHUNK, _CHUNK)],
          hsem[b % 2])

    def wait_store(b):
      pltpu.make_async_copy(
          stage.at[sid, b % 2], out_hbm.at[pl.ds(0, _CHUNK)], hsem[b % 2]).wait()

    for b in range(_NBUF):
      start_gather(b, b)

    def body(go, carry):
      for b in range(_NBUF):
        g = go * _NBUF + b
        wait_gather(b)
        @pl.when((go > 0) | (b >= 2))
        def _():
          wait_store(b)
        start_stage(b)
        wait_stage(b)
        start_store(g, b)

        @pl.when(go < (n_chunks // _NBUF) - 1)
        def _():
          start_gather(g + _NBUF, b)
      return carry

    lax.fori_loop(0, n_chunks // _NBUF, body, 0)
    for b in range(2):
      wait_store(b)

  return gather_kernel


def kernel(input_ids, input_mask, table):
  del input_mask
  b, s = input_ids.shape
  vocab, dim = table.shape
  ids_flat = input_ids.reshape(-1).astype(jnp.int32)
  out = _make_gather(b * s, dim)(ids_flat, table)
  return out.reshape(b, s, dim)
